# baseline (speedup 1.0000x reference)
import functools

import jax
import jax.numpy as jnp
import numpy as np
from jax.experimental import pallas as pl
from jax.experimental.pallas import tpu as pltpu

MAX_LEN = 2048
D_PE = 16
N_TS = D_PE // 2


def _np_pe_tables(V):
    ts = (MAX_LEN ** np.linspace(0.0, 1.0, N_TS)).astype(np.float32)
    ts_row = np.repeat(ts, 16).reshape(1, N_TS * 16).astype(np.float32)
    scaled = np.arange(V, dtype=np.float32)[:, None] / ts[None, :]
    var_pe = np.concatenate([np.sin(scaled), np.cos(scaled)], axis=1)
    return ts_row, var_pe.astype(np.float32)


def _seft_body(src_ref, times_ref, static_ref, tsrow_ref, wv_ref, bv_ref,
               varpe_ref, wsum_ref, bmap_ref, wembT_ref, bemb_ref,
               wm1T_ref, bm1_ref, wm2T_ref, bm2_ref, out_ref,
               acc_sin, acc_cos, colcnt, colfea):
    i = pl.program_id(0)

    @pl.when(i == 0)
    def _init():
        acc_sin[...] = jnp.zeros_like(acc_sin)
        acc_cos[...] = jnp.zeros_like(acc_cos)
        colcnt[...] = jnp.zeros_like(colcnt)
        colfea[...] = jnp.zeros_like(colfea)

    fea = src_ref[:, :, :src_ref.shape[2] // 2]
    mask = (fea != 0.0).astype(jnp.float32)
    colcnt[...] += jnp.sum(mask, axis=0)
    colfea[...] += jnp.sum(fea, axis=0)
    rowcnt = jnp.sum(mask, axis=2)

    tb = times_ref[...]
    t_big = jnp.concatenate([tb] * N_TS, axis=1) / tsrow_ref[...]
    rc8 = jnp.concatenate([rowcnt] * N_TS, axis=1)
    acc_sin[...] += jnp.sum(jnp.sin(t_big) * rc8, axis=0, keepdims=True)
    acc_cos[...] += jnp.sum(jnp.cos(t_big) * rc8, axis=0, keepdims=True)

    @pl.when(i == pl.num_programs(0) - 1)
    def _epilogue():
        f32 = jnp.float32
        cc = colcnt[...]
        count = jnp.sum(cc, axis=1, keepdims=True)
        sumfea = jnp.sum(colfea[...], axis=1, keepdims=True)
        denom = jnp.maximum(count, 1.0)

        asin = acc_sin[...]
        acosv = acc_cos[...]
        rows = [asin[0:1, 16 * d:16 * (d + 1)] for d in range(N_TS)]
        rows += [acosv[0:1, 16 * d:16 * (d + 1)] for d in range(N_TS)]
        spe_t = jnp.concatenate(rows, axis=0)

        w_pe = wsum_ref[0:16, :]
        w_val = wsum_ref[16:32, :]
        w_var = wsum_ref[32:48, :]
        term_pe = jax.lax.dot_general(
            spe_t, w_pe, (((0,), (0,)), ((), ())), preferred_element_type=f32,
            precision=jax.lax.Precision.HIGHEST)
        sum_val = (jnp.dot(sumfea, wv_ref[...], preferred_element_type=f32, precision=jax.lax.Precision.HIGHEST)
                   + jnp.dot(count, bv_ref[...], preferred_element_type=f32, precision=jax.lax.Precision.HIGHEST))
        term_val = jnp.dot(sum_val, w_val, preferred_element_type=f32, precision=jax.lax.Precision.HIGHEST)
        sum_var = jnp.dot(cc, varpe_ref[...], preferred_element_type=f32, precision=jax.lax.Precision.HIGHEST)
        term_var = jnp.dot(sum_var, w_var, preferred_element_type=f32, precision=jax.lax.Precision.HIGHEST)

        raw = term_pe + term_val + term_var
        out128 = jnp.where(count > 0, raw / denom, 0.0) + bmap_ref[...]
        emb = jnp.dot(static_ref[...], wembT_ref[...],
                      preferred_element_type=f32, precision=jax.lax.Precision.HIGHEST) + bemb_ref[...]
        cat = jnp.concatenate([out128, emb], axis=1)
        h = jnp.maximum(
            jnp.dot(cat, wm1T_ref[...], preferred_element_type=f32, precision=jax.lax.Precision.HIGHEST)
            + bm1_ref[...], 0.0)
        out_ref[...] = jnp.dot(h, wm2T_ref[...],
                               preferred_element_type=f32, precision=jax.lax.Precision.HIGHEST) + bm2_ref[...]


@functools.partial(jax.jit, static_argnames=())
def _seft(src, static, times, W_value, b_value, W_map, b_map, W_emb, b_emb,
          W_mlp1, b_mlp1, W_mlp2, b_mlp2):
    T, B = src.shape[0], src.shape[1]
    V = src.shape[2] // 2
    TB = 256
    grid = T // TB

    ts_row_np, var_pe_np = _np_pe_tables(V)
    ts_row = jnp.asarray(ts_row_np)
    var_pe = jnp.asarray(var_pe_np)
    wsum_t = (W_map[:, :3 * D_PE] + W_map[:, 3 * D_PE:]).T

    full = lambda shape: pl.BlockSpec(shape, lambda i: tuple(0 for _ in shape))
    operands = (
        src, times, static, ts_row,
        W_value.reshape(1, 16), b_value.reshape(1, 16),
        var_pe, wsum_t, b_map.reshape(1, -1),
        W_emb.T, b_emb.reshape(1, -1),
        W_mlp1.T, b_mlp1.reshape(1, -1),
        W_mlp2.T, b_mlp2.reshape(1, -1),
    )
    in_specs = [
        pl.BlockSpec((TB, B, 2 * V), lambda i: (i, 0, 0)),
        pl.BlockSpec((TB, B), lambda i: (i, 0)),
    ] + [full(op.shape) for op in operands[2:]]

    return pl.pallas_call(
        _seft_body,
        grid=(grid,),
        in_specs=in_specs,
        out_specs=pl.BlockSpec((B, 2), lambda i: (0, 0)),
        out_shape=jax.ShapeDtypeStruct((B, 2), jnp.float32),
        scratch_shapes=[
            pltpu.VMEM((1, 128), jnp.float32),
            pltpu.VMEM((1, 128), jnp.float32),
            pltpu.VMEM((B, V), jnp.float32),
            pltpu.VMEM((B, V), jnp.float32),
        ],
        compiler_params=pltpu.CompilerParams(
            dimension_semantics=("arbitrary",)),
    )(*operands)


def kernel(src, static, times, lengths, W_value, b_value, W_map, b_map,
           W_emb, b_emb, W_mlp1, b_mlp1, W_mlp2, b_mlp2):
    del lengths
    return _seft(src, static, times, W_value, b_value, W_map, b_map,
                 W_emb, b_emb, W_mlp1, b_mlp1, W_mlp2, b_mlp2)


# speedup vs baseline: 1.4262x; 1.4262x over previous
import functools

import jax
import jax.numpy as jnp
import numpy as np
from jax.experimental import pallas as pl
from jax.experimental.pallas import tpu as pltpu

MAX_LEN = 2048
D_PE = 16
N_TS = D_PE // 2
HIGHEST = jax.lax.Precision.HIGHEST


def _np_tables(V):
    W = 2 * V
    ts = (MAX_LEN ** np.linspace(0.0, 1.0, N_TS)).astype(np.float32)
    ts_row = np.repeat(ts, 16).reshape(1, N_TS * 16).astype(np.float32)
    scaled = np.arange(V, dtype=np.float32)[:, None] / ts[None, :]
    var_pe = np.concatenate([np.sin(scaled), np.cos(scaled)], axis=1)

    j = np.arange(16 * W)
    feamask = (j % W < V).astype(np.float32)
    sel = (j // W == np.arange(16)[:, None]).astype(np.float32)
    s_bf = (sel * feamask).T.astype(np.float32)
    gv = np.zeros((16 * W, D_PE), np.float32)
    gv[feamask > 0, :] = var_pe[j[feamask > 0] % W, :]
    ones_fea = feamask.reshape(-1, 1)
    return (ts_row, s_bf.astype(np.float32), sel, gv, ones_fea)


def _seft_body(src_ref, times_ref, static_ref, tsrow_ref, s_ref, r_ref,
               gv_ref, onesfea_ref, wv_ref, bv_ref, wsum_ref, bmap_ref,
               wembT_ref, bemb_ref, wm1T_ref, bm1_ref, wm2T_ref, bm2_ref,
               out_ref, acc_sin, acc_cos, colcnt, colraw):
    i = pl.program_id(0)
    f32 = jnp.float32
    bf16 = jnp.bfloat16

    @pl.when(i == 0)
    def _init():
        acc_sin[...] = jnp.zeros_like(acc_sin)
        acc_cos[...] = jnp.zeros_like(acc_cos)
        colcnt[...] = jnp.zeros_like(colcnt)
        colraw[...] = jnp.zeros_like(colraw)

    x = src_ref[...]
    obs = (x != 0.0).astype(bf16)
    ones_tb = jnp.ones((1, x.shape[0]), bf16)
    rowcnt = jax.lax.dot_general(
        obs, s_ref[...].astype(bf16), (((1,), (0,)), ((), ())),
        preferred_element_type=f32)
    colcnt[...] += jax.lax.dot_general(
        ones_tb, obs, (((1,), (0,)), ((), ())), preferred_element_type=f32)
    colraw[...] += jnp.sum(x, axis=0, keepdims=True)

    tb = times_ref[...]
    t_big = jnp.concatenate([tb] * N_TS, axis=1) / tsrow_ref[...]
    rc8 = jnp.concatenate([rowcnt] * N_TS, axis=1)
    acc_sin[...] += jnp.sum(jnp.sin(t_big) * rc8, axis=0, keepdims=True)
    acc_cos[...] += jnp.sum(jnp.cos(t_big) * rc8, axis=0, keepdims=True)

    @pl.when(i == pl.num_programs(0) - 1)
    def _epilogue():
        colmat = r_ref[...] * colcnt[...]
        crmat = r_ref[...] * colraw[...]
        count = jnp.dot(colmat, onesfea_ref[...],
                        preferred_element_type=f32, precision=HIGHEST)
        sumfea = jnp.dot(crmat, onesfea_ref[...],
                         preferred_element_type=f32, precision=HIGHEST)
        denom = jnp.maximum(count, 1.0)
        sum_var = jnp.dot(colmat, gv_ref[...],
                          preferred_element_type=f32, precision=HIGHEST)

        asin = acc_sin[...]
        acosv = acc_cos[...]
        rows = [asin[0:1, 16 * d:16 * (d + 1)] for d in range(N_TS)]
        rows += [acosv[0:1, 16 * d:16 * (d + 1)] for d in range(N_TS)]
        spe_t = jnp.concatenate(rows, axis=0)

        w_pe = wsum_ref[0:16, :]
        w_val = wsum_ref[16:32, :]
        w_var = wsum_ref[32:48, :]
        term_pe = jax.lax.dot_general(
            spe_t, w_pe, (((0,), (0,)), ((), ())), preferred_element_type=f32,
            precision=HIGHEST)
        sum_val = (jnp.dot(sumfea, wv_ref[...],
                           preferred_element_type=f32, precision=HIGHEST)
                   + jnp.dot(count, bv_ref[...],
                             preferred_element_type=f32, precision=HIGHEST))
        term_val = jnp.dot(sum_val, w_val,
                           preferred_element_type=f32, precision=HIGHEST)
        term_var = jnp.dot(sum_var, w_var,
                           preferred_element_type=f32, precision=HIGHEST)

        raw = term_pe + term_val + term_var
        out128 = jnp.where(count > 0, raw / denom, 0.0) + bmap_ref[...]
        emb = jnp.dot(static_ref[...], wembT_ref[...],
                      preferred_element_type=f32, precision=HIGHEST)
        emb = emb + bemb_ref[...]
        cat = jnp.concatenate([out128, emb], axis=1)
        h = jnp.maximum(
            jnp.dot(cat, wm1T_ref[...],
                    preferred_element_type=f32, precision=HIGHEST)
            + bm1_ref[...], 0.0)
        out_ref[...] = jnp.dot(h, wm2T_ref[...],
                               preferred_element_type=f32,
                               precision=HIGHEST) + bm2_ref[...]


@functools.partial(jax.jit, static_argnames=())
def _seft(src, static, times, W_value, b_value, W_map, b_map, W_emb, b_emb,
          W_mlp1, b_mlp1, W_mlp2, b_mlp2):
    T, B = src.shape[0], src.shape[1]
    V = src.shape[2] // 2
    TB = 256
    grid = T // TB
    src_flat = src.reshape(T, B * 2 * V)

    ts_row, s_sel, r_sel, gv, ones_fea = map(jnp.asarray, _np_tables(V))
    wsum_t = (W_map[:, :3 * D_PE] + W_map[:, 3 * D_PE:]).T

    full = lambda shape: pl.BlockSpec(shape, lambda i: tuple(0 for _ in shape))
    operands = (
        src_flat, times, static, ts_row, s_sel, r_sel, gv, ones_fea,
        W_value.reshape(1, 16), b_value.reshape(1, 16),
        wsum_t, b_map.reshape(1, -1),
        W_emb.T, b_emb.reshape(1, -1),
        W_mlp1.T, b_mlp1.reshape(1, -1),
        W_mlp2.T, b_mlp2.reshape(1, -1),
    )
    in_specs = [
        pl.BlockSpec((TB, B * 2 * V), lambda i: (i, 0)),
        pl.BlockSpec((TB, B), lambda i: (i, 0)),
    ] + [full(op.shape) for op in operands[2:]]

    return pl.pallas_call(
        _seft_body,
        grid=(grid,),
        in_specs=in_specs,
        out_specs=pl.BlockSpec((B, 2), lambda i: (0, 0)),
        out_shape=jax.ShapeDtypeStruct((B, 2), jnp.float32),
        scratch_shapes=[
            pltpu.VMEM((1, 128), jnp.float32),
            pltpu.VMEM((1, 128), jnp.float32),
            pltpu.VMEM((1, B * 2 * V), jnp.float32),
            pltpu.VMEM((1, B * 2 * V), jnp.float32),
        ],
        compiler_params=pltpu.CompilerParams(
            dimension_semantics=("arbitrary",)),
    )(*operands)


def kernel(src, static, times, lengths, W_value, b_value, W_map, b_map,
           W_emb, b_emb, W_mlp1, b_mlp1, W_mlp2, b_mlp2):
    del lengths
    return _seft(src, static, times, W_value, b_value, W_map, b_map,
                 W_emb, b_emb, W_mlp1, b_mlp1, W_mlp2, b_mlp2)
